# Initial kernel scaffold; baseline (speedup 1.0000x reference)
import jax
import jax.numpy as jnp
from jax.experimental import pallas as pl


def kernel(x, table):
    raise NotImplementedError("write your pallas kernel here")


# speedup vs baseline: 1.8312x; 1.8312x over previous
import functools

import jax
import jax.numpy as jnp
from jax import lax
from jax.experimental import pallas as pl
from jax.experimental.pallas import tpu as pltpu
from jax.experimental.pallas import tpu_sc as plsc

_NC = 2
_NS = 16
_NW = _NC * _NS


@functools.partial(jax.jit, static_argnames=("b_per_w", "chunk"))
def _sc_gather(idx, table, b_per_w, chunk):
    B = idx.shape[0]
    D = table.shape[1]
    n_chunks = b_per_w // chunk
    mesh = plsc.VectorSubcoreMesh(core_axis_name="c", subcore_axis_name="s")

    @functools.partial(
        pl.kernel,
        out_type=jax.ShapeDtypeStruct((B, D), jnp.float32),
        mesh=mesh,
        scratch_types=[
            pltpu.VMEM((chunk,), jnp.int32),
            pltpu.VMEM((chunk, D), jnp.float32),
            pltpu.SemaphoreType.DMA,
        ],
        compiler_params=pltpu.CompilerParams(use_tc_tiling_on_sc=False),
    )
    def k(idx_hbm, table_hbm, out_hbm, idx_v, rows_v, sem):
        wid = lax.axis_index("s") * _NC + lax.axis_index("c")
        base = wid * b_per_w

        def body(i, _):
            off = base + i * chunk
            pltpu.sync_copy(idx_hbm.at[pl.ds(off, chunk)], idx_v)
            pltpu.async_copy(table_hbm.at[idx_v], rows_v, sem).wait()
            pltpu.sync_copy(rows_v, out_hbm.at[pl.ds(off, chunk)])
            return 0

        lax.fori_loop(0, n_chunks, body, 0)

    return k(idx, table)


def kernel(x, table):
    B = x.size
    idx = x.reshape(B).astype(jnp.int32)
    b_per_w = B // _NW
    out = _sc_gather(idx, table, b_per_w, 800)
    return out.reshape(*x.shape, table.shape[1])


# speedup vs baseline: 1.8590x; 1.0152x over previous
import functools

import jax
import jax.numpy as jnp
from jax import lax
from jax.experimental import pallas as pl
from jax.experimental.pallas import tpu as pltpu
from jax.experimental.pallas import tpu_sc as plsc

_NC = 2
_NS = 16
_NW = _NC * _NS


@functools.partial(jax.jit, static_argnames=("b_per_w", "chunk", "nbuf"))
def _sc_gather(idx, table, b_per_w, chunk, nbuf):
    B = idx.shape[0]
    D = table.shape[1]
    n = b_per_w // chunk
    assert n % nbuf == 0 and n >= nbuf
    mesh = plsc.VectorSubcoreMesh(core_axis_name="c", subcore_axis_name="s")

    @functools.partial(
        pl.kernel,
        out_type=jax.ShapeDtypeStruct((B, D), jnp.float32),
        mesh=mesh,
        scratch_types=(
            [pltpu.VMEM((nbuf, chunk), jnp.int32),
             pltpu.VMEM((nbuf, chunk, D), jnp.float32)]
            + [pltpu.SemaphoreType.DMA] * (2 * nbuf)
        ),
        compiler_params=pltpu.CompilerParams(use_tc_tiling_on_sc=False),
    )
    def k(idx_hbm, table_hbm, out_hbm, idx_v, rows_v, *sems):
        gs, ws = sems[:nbuf], sems[nbuf:]
        wid = lax.axis_index("s") * _NC + lax.axis_index("c")
        base = wid * b_per_w

        def fire_gather(i, b):
            off = base + i * chunk
            pltpu.sync_copy(idx_hbm.at[pl.ds(off, chunk)], idx_v.at[b])
            pltpu.async_copy(table_hbm.at[idx_v.at[b]], rows_v.at[b], gs[b])

        for b in range(nbuf - 1):
            fire_gather(jnp.int32(b), b)

        @pl.loop(0, n, step=nbuf)
        def _outer(i0):
            for b in range(nbuf):
                i = i0 + b
                sp = (b + nbuf - 1) % nbuf
                pre = i + nbuf - 1

                @pl.when(jnp.logical_and(pre < n, i >= 1))
                def _():
                    pltpu.make_async_copy(
                        rows_v.at[sp], out_hbm.at[pl.ds(base, chunk)], ws[sp]
                    ).wait()

                @pl.when(pre < n)
                def _():
                    fire_gather(pre, sp)

                pltpu.make_async_copy(
                    table_hbm.at[idx_v.at[b]], rows_v.at[b], gs[b]
                ).wait()
                off = base + i * chunk
                pltpu.async_copy(rows_v.at[b], out_hbm.at[pl.ds(off, chunk)], ws[b])

        for b in range(nbuf):
            pltpu.make_async_copy(
                rows_v.at[b], out_hbm.at[pl.ds(base, chunk)], ws[b]
            ).wait()

    return k(idx, table)


def kernel(x, table):
    B = x.size
    idx = x.reshape(B).astype(jnp.int32)
    b_per_w = B // _NW
    out = _sc_gather(idx, table, b_per_w, 400, 4)
    return out.reshape(*x.shape, table.shape[1])
